# Initial kernel scaffold; baseline (speedup 1.0000x reference)
import jax
import jax.numpy as jnp
from jax.experimental import pallas as pl


def kernel(embeds, tag_ids, lengths, h0, c0, w_ih_f, w_hh_f, b_ih_f, b_hh_f, w_ih_b, w_hh_b, b_ih_b, b_hh_b, w_out, b_out, start_trans, end_trans, trans):
    raise NotImplementedError("write your pallas kernel here")


# speedup vs baseline: 5.8154x; 5.8154x over previous
import jax
import jax.numpy as jnp
from jax.experimental import pallas as pl
from jax.experimental.pallas import tpu as pltpu

_T = 512
_B = 64
_E = 1024
_H = 512
_G = 4 * _H
_K = 74
_KP = 128
_BH = _B // 2
_NEG = -1e30


def _lstm_body(x_ref, wih_ref, whh_ref, b_ref, wo_ref, h0_ref, c0_ref,
               em_ref, h_scr, c_scr):
    t = pl.program_id(1)

    @pl.when(t == 0)
    def _():
        h_scr[...] = h0_ref[0]
        c_scr[...] = c0_ref[0]

    h_prev = h_scr[...]
    g = jnp.dot(x_ref[0], wih_ref[0], preferred_element_type=jnp.float32)
    g = g + jnp.dot(h_prev.astype(jnp.bfloat16), whh_ref[0],
                    preferred_element_type=jnp.float32)
    g = g + b_ref[0]
    i = jax.nn.sigmoid(g[:, 0:_H])
    f = jax.nn.sigmoid(g[:, _H:2 * _H])
    gg = jnp.tanh(g[:, 2 * _H:3 * _H])
    o = jax.nn.sigmoid(g[:, 3 * _H:4 * _H])
    c = f * c_scr[...] + i * gg
    h = o * jnp.tanh(c)
    c_scr[...] = c
    h_scr[...] = h
    em_ref[0, 0] = jnp.dot(h.astype(jnp.bfloat16), wo_ref[0],
                           preferred_element_type=jnp.float32)


def _crf_body(emf_ref, emb_ref, tags_ref, lens_ref, bout_ref, start_ref,
              end_ref, transn_ref, transz_ref, out_ref,
              alpha_scr, acc_scr, poh_scr, expt_scr):
    t = pl.program_id(1)

    em_t = emf_ref[0, 0] + emb_ref[0, 0] + bout_ref[...]
    lens = lens_ref[...]
    lanes = jax.lax.broadcasted_iota(jnp.int32, (_BH, _KP), 1)
    oh = (tags_ref[0] == lanes).astype(jnp.float32)
    endv = end_ref[...]

    @pl.when(t == 0)
    def _():
        expt_scr[...] = jnp.exp(transn_ref[...]).astype(jnp.bfloat16)
        alpha_scr[...] = start_ref[...] + em_t
        acc_scr[...] = oh * (em_t + start_ref[...])
        poh_scr[...] = oh.astype(jnp.bfloat16)

    @pl.when(t > 0)
    def _():
        a = alpha_scr[...]
        mx = jnp.max(a, axis=-1, keepdims=True)
        p = jnp.exp(a - mx)
        s = jnp.dot(p.astype(jnp.bfloat16), expt_scr[...],
                    preferred_element_type=jnp.float32)
        anew = jnp.log(s) + mx + em_t
        alpha_scr[...] = jnp.where(t < lens, anew, a)
        trow = jnp.dot(poh_scr[...], transz_ref[...],
                       preferred_element_type=jnp.float32)
        acc_scr[...] = acc_scr[...] + (t < lens).astype(jnp.float32) * oh * (em_t + trow)
        poh_scr[...] = oh.astype(jnp.bfloat16)

    acc_scr[...] = acc_scr[...] + jnp.where(lens == t + 1, oh * endv, 0.0)

    @pl.when(t == _T - 1)
    def _():
        a = alpha_scr[...] + endv
        mx = jnp.max(a, axis=-1, keepdims=True)
        lz = jnp.log(jnp.sum(jnp.exp(a - mx), axis=-1, keepdims=True)) + mx
        numer = jnp.sum(acc_scr[...], axis=-1, keepdims=True)
        total = jnp.sum(numer - lz)
        out_ref[...] = jnp.broadcast_to(total, (1, 1, _KP))


def kernel(embeds, tag_ids, lengths, h0, c0, w_ih_f, w_hh_f, b_ih_f, b_hh_f,
           w_ih_b, w_hh_b, b_ih_b, b_hh_b, w_out, b_out,
           start_trans, end_trans, trans):
    f32 = jnp.float32
    bf16 = jnp.bfloat16

    xT = jnp.transpose(embeds, (1, 0, 2)).astype(bf16)
    wih = jnp.stack([w_ih_f.T, w_ih_b.T]).astype(bf16)
    whh = jnp.stack([w_hh_f.T, w_hh_b.T]).astype(bf16)
    bias = jnp.stack([b_ih_f + b_hh_f, b_ih_b + b_hh_b])[:, None, :]
    wo = jnp.zeros((2, _H, _KP), f32)
    wo = wo.at[0, :, :_K].set(w_out[:, :_H].T)
    wo = wo.at[1, :, :_K].set(w_out[:, _H:].T)
    wo = wo.astype(bf16)

    rev = lambda d, t: jnp.where(d == 0, t, _T - 1 - t)

    em = pl.pallas_call(
        _lstm_body,
        grid=(2, _T),
        in_specs=[
            pl.BlockSpec((1, _B, _E), lambda d, t: (rev(d, t), 0, 0)),
            pl.BlockSpec((1, _E, _G), lambda d, t: (d, 0, 0)),
            pl.BlockSpec((1, _H, _G), lambda d, t: (d, 0, 0)),
            pl.BlockSpec((1, 1, _G), lambda d, t: (d, 0, 0)),
            pl.BlockSpec((1, _H, _KP), lambda d, t: (d, 0, 0)),
            pl.BlockSpec((1, _B, _H), lambda d, t: (d, 0, 0)),
            pl.BlockSpec((1, _B, _H), lambda d, t: (d, 0, 0)),
        ],
        out_specs=pl.BlockSpec((1, 1, _B, _KP),
                               lambda d, t: (d, rev(d, t), 0, 0)),
        out_shape=jax.ShapeDtypeStruct((2, _T, _B, _KP), f32),
        scratch_shapes=[pltpu.VMEM((_B, _H), f32), pltpu.VMEM((_B, _H), f32)],
        compiler_params=pltpu.CompilerParams(
            dimension_semantics=("parallel", "arbitrary")),
        name="bilstm_em",
    )(xT, wih, whh, bias, wo, h0, c0)

    tags_b = jnp.broadcast_to(tag_ids.T[:, :, None], (_T, _B, _KP))
    lens_b = jnp.broadcast_to(lengths[:, None], (_B, _KP))
    bout_p = jnp.full((1, _KP), _NEG, f32).at[0, :_K].set(b_out)
    start_p = jnp.full((1, _KP), _NEG, f32).at[0, :_K].set(start_trans)
    end_p = jnp.full((1, _KP), _NEG, f32).at[0, :_K].set(end_trans)
    trans_n = jnp.full((_KP, _KP), _NEG, f32).at[:_K, :_K].set(trans)
    trans_z = jnp.zeros((_KP, _KP), bf16).at[:_K, :_K].set(trans.astype(bf16))

    partial = pl.pallas_call(
        _crf_body,
        grid=(2, _T),
        in_specs=[
            pl.BlockSpec((1, 1, _BH, _KP), lambda p, t: (0, t, p, 0)),
            pl.BlockSpec((1, 1, _BH, _KP), lambda p, t: (1, t, p, 0)),
            pl.BlockSpec((1, _BH, _KP), lambda p, t: (t, p, 0)),
            pl.BlockSpec((_BH, _KP), lambda p, t: (p, 0)),
            pl.BlockSpec((1, _KP), lambda p, t: (0, 0)),
            pl.BlockSpec((1, _KP), lambda p, t: (0, 0)),
            pl.BlockSpec((1, _KP), lambda p, t: (0, 0)),
            pl.BlockSpec((_KP, _KP), lambda p, t: (0, 0)),
            pl.BlockSpec((_KP, _KP), lambda p, t: (0, 0)),
        ],
        out_specs=pl.BlockSpec((1, 1, _KP), lambda p, t: (p, 0, 0)),
        out_shape=jax.ShapeDtypeStruct((2, 1, _KP), f32),
        scratch_shapes=[
            pltpu.VMEM((_BH, _KP), f32),
            pltpu.VMEM((_BH, _KP), f32),
            pltpu.VMEM((_BH, _KP), bf16),
            pltpu.VMEM((_KP, _KP), bf16),
        ],
        compiler_params=pltpu.CompilerParams(
            dimension_semantics=("parallel", "arbitrary")),
        name="crf_nll",
    )(em, em, tags_b, lens_b, bout_p, start_p, end_p, trans_n, trans_z)

    return -(partial[0, 0, 0] + partial[1, 0, 0]) / _B


# speedup vs baseline: 10.4320x; 1.7939x over previous
import jax
import jax.numpy as jnp
from jax.experimental import pallas as pl
from jax.experimental.pallas import tpu as pltpu

_T = 512
_B = 64
_E = 1024
_H = 512
_G = 4 * _H
_K = 74
_KP = 128
_BH = _B // 2
_C = 8
_TC = _T // _C
_NEG = -1e30


def _lstm_body(x_ref, wih_ref, whh_ref, b_ref, wo_ref, h0_ref, c0_ref,
               em_ref, h_scr, c_scr, gx_scr):
    d = pl.program_id(0)
    c = pl.program_id(1)

    @pl.when(c == 0)
    def _():
        h_scr[...] = h0_ref[0]
        c_scr[...] = c0_ref[0]

    xc = x_ref[0].reshape(_C * _B, _E)
    gx_scr[...] = jnp.dot(xc, wih_ref[0], preferred_element_type=jnp.float32)

    for k in range(_C):
        idx = jnp.where(d == 0, k, _C - 1 - k)
        row = pl.multiple_of(idx * _B, _B)
        g = gx_scr[pl.ds(row, _B), :]
        g = g + jnp.dot(h_scr[...].astype(jnp.bfloat16), whh_ref[0],
                        preferred_element_type=jnp.float32)
        g = g + b_ref[0]
        i = jax.nn.sigmoid(g[:, 0:_H])
        f = jax.nn.sigmoid(g[:, _H:2 * _H])
        gg = jnp.tanh(g[:, 2 * _H:3 * _H])
        o = jax.nn.sigmoid(g[:, 3 * _H:4 * _H])
        cc = f * c_scr[...] + i * gg
        h = o * jnp.tanh(cc)
        c_scr[...] = cc
        h_scr[...] = h
        em_ref[0, idx] = jnp.dot(h.astype(jnp.bfloat16), wo_ref[0],
                                 preferred_element_type=jnp.float32)


def _crf_body(emf_ref, emb_ref, tags_ref, lens_ref, bout_ref, start_ref,
              end_ref, transn_ref, transz_ref, out_ref,
              alpha_scr, acc_scr, poh_scr, expt_scr):
    c = pl.program_id(1)
    lens = lens_ref[...]
    lanes = jax.lax.broadcasted_iota(jnp.int32, (_BH, _KP), 1)
    endv = end_ref[...]

    for k in range(_C):
        t = c * _C + k
        em_t = emf_ref[0, k] + emb_ref[0, k] + bout_ref[...]
        oh = (tags_ref[k] == lanes).astype(jnp.float32)

        def _update():
            a = alpha_scr[...]
            mx = jnp.max(a, axis=-1, keepdims=True)
            p = jnp.exp(a - mx)
            s = jnp.dot(p.astype(jnp.bfloat16), expt_scr[...],
                        preferred_element_type=jnp.float32)
            anew = jnp.log(s) + mx + em_t
            alpha_scr[...] = jnp.where(t < lens, anew, a)
            trow = jnp.dot(poh_scr[...], transz_ref[...],
                           preferred_element_type=jnp.float32)
            acc_scr[...] = acc_scr[...] + (t < lens).astype(jnp.float32) * oh * (em_t + trow)
            poh_scr[...] = oh.astype(jnp.bfloat16)

        if k == 0:
            @pl.when(c == 0)
            def _():
                expt_scr[...] = jnp.exp(transn_ref[...]).astype(jnp.bfloat16)
                alpha_scr[...] = start_ref[...] + em_t
                acc_scr[...] = oh * (em_t + start_ref[...])
                poh_scr[...] = oh.astype(jnp.bfloat16)

            pl.when(c > 0)(_update)
        else:
            _update()

        acc_scr[...] = acc_scr[...] + jnp.where(lens == t + 1, oh * endv, 0.0)

    @pl.when(c == _TC - 1)
    def _():
        a = alpha_scr[...] + endv
        mx = jnp.max(a, axis=-1, keepdims=True)
        lz = jnp.log(jnp.sum(jnp.exp(a - mx), axis=-1, keepdims=True)) + mx
        numer = jnp.sum(acc_scr[...], axis=-1, keepdims=True)
        total = jnp.sum(numer - lz)
        out_ref[...] = jnp.broadcast_to(total, (1, 1, _KP))


def kernel(embeds, tag_ids, lengths, h0, c0, w_ih_f, w_hh_f, b_ih_f, b_hh_f,
           w_ih_b, w_hh_b, b_ih_b, b_hh_b, w_out, b_out,
           start_trans, end_trans, trans):
    f32 = jnp.float32
    bf16 = jnp.bfloat16

    xT = jnp.transpose(embeds, (1, 0, 2)).astype(bf16)
    xT = xT.reshape(_TC, _C, _B, _E)
    wih = jnp.stack([w_ih_f.T, w_ih_b.T]).astype(bf16)
    whh = jnp.stack([w_hh_f.T, w_hh_b.T]).astype(bf16)
    bias = jnp.stack([b_ih_f + b_hh_f, b_ih_b + b_hh_b])[:, None, :]
    wo = jnp.zeros((2, _H, _KP), f32)
    wo = wo.at[0, :, :_K].set(w_out[:, :_H].T)
    wo = wo.at[1, :, :_K].set(w_out[:, _H:].T)
    wo = wo.astype(bf16)

    rev = lambda d, c: jnp.where(d == 0, c, _TC - 1 - c)

    em = pl.pallas_call(
        _lstm_body,
        grid=(2, _TC),
        in_specs=[
            pl.BlockSpec((1, _C, _B, _E), lambda d, c: (rev(d, c), 0, 0, 0)),
            pl.BlockSpec((1, _E, _G), lambda d, c: (d, 0, 0)),
            pl.BlockSpec((1, _H, _G), lambda d, c: (d, 0, 0)),
            pl.BlockSpec((1, 1, _G), lambda d, c: (d, 0, 0)),
            pl.BlockSpec((1, _H, _KP), lambda d, c: (d, 0, 0)),
            pl.BlockSpec((1, _B, _H), lambda d, c: (d, 0, 0)),
            pl.BlockSpec((1, _B, _H), lambda d, c: (d, 0, 0)),
        ],
        out_specs=pl.BlockSpec((1, _C, _B, _KP),
                               lambda d, c: (d, rev(d, c), 0, 0)),
        out_shape=jax.ShapeDtypeStruct((2, _TC * _C, _B, _KP), f32),
        scratch_shapes=[pltpu.VMEM((_B, _H), f32), pltpu.VMEM((_B, _H), f32),
                        pltpu.VMEM((_C * _B, _G), f32)],
        compiler_params=pltpu.CompilerParams(
            dimension_semantics=("parallel", "arbitrary")),
        name="bilstm_em",
    )(xT, wih, whh, bias, wo, h0, c0)

    tags_b = jnp.broadcast_to(tag_ids.T[:, :, None], (_T, _B, _KP))
    lens_b = jnp.broadcast_to(lengths[:, None], (_B, _KP))
    bout_p = jnp.full((1, _KP), _NEG, f32).at[0, :_K].set(b_out)
    start_p = jnp.full((1, _KP), _NEG, f32).at[0, :_K].set(start_trans)
    end_p = jnp.full((1, _KP), _NEG, f32).at[0, :_K].set(end_trans)
    trans_n = jnp.full((_KP, _KP), _NEG, f32).at[:_K, :_K].set(trans)
    trans_z = jnp.zeros((_KP, _KP), bf16).at[:_K, :_K].set(trans.astype(bf16))

    partial = pl.pallas_call(
        _crf_body,
        grid=(2, _TC),
        in_specs=[
            pl.BlockSpec((1, _C, _BH, _KP), lambda p, c: (0, c, p, 0)),
            pl.BlockSpec((1, _C, _BH, _KP), lambda p, c: (1, c, p, 0)),
            pl.BlockSpec((_C, _BH, _KP), lambda p, c: (c, p, 0)),
            pl.BlockSpec((_BH, _KP), lambda p, c: (p, 0)),
            pl.BlockSpec((1, _KP), lambda p, c: (0, 0)),
            pl.BlockSpec((1, _KP), lambda p, c: (0, 0)),
            pl.BlockSpec((1, _KP), lambda p, c: (0, 0)),
            pl.BlockSpec((_KP, _KP), lambda p, c: (0, 0)),
            pl.BlockSpec((_KP, _KP), lambda p, c: (0, 0)),
        ],
        out_specs=pl.BlockSpec((1, 1, _KP), lambda p, c: (p, 0, 0)),
        out_shape=jax.ShapeDtypeStruct((2, 1, _KP), f32),
        scratch_shapes=[
            pltpu.VMEM((_BH, _KP), f32),
            pltpu.VMEM((_BH, _KP), f32),
            pltpu.VMEM((_BH, _KP), bf16),
            pltpu.VMEM((_KP, _KP), bf16),
        ],
        compiler_params=pltpu.CompilerParams(
            dimension_semantics=("parallel", "arbitrary")),
        name="crf_nll",
    )(em, em, tags_b, lens_b, bout_p, start_p, end_p, trans_n, trans_z)

    return -(partial[0, 0, 0] + partial[1, 0, 0]) / _B


# speedup vs baseline: 12.0372x; 1.1539x over previous
import jax
import jax.numpy as jnp
from jax.experimental import pallas as pl
from jax.experimental.pallas import tpu as pltpu

_T = 512
_B = 64
_E = 1024
_H = 512
_G = 4 * _H
_K = 74
_KP = 128
_C = 8
_TC = _T // _C
_NEG = -1e30


def _cell(gx_scr, row, h, c, whh_ref, b_ref, wo_ref, d):
    g = gx_scr[row:row + _B, :]
    g = g + jnp.dot(h.astype(jnp.bfloat16), whh_ref[d],
                    preferred_element_type=jnp.float32)
    g = g + b_ref[d]
    i = jax.nn.sigmoid(g[:, 0:_H])
    f = jax.nn.sigmoid(g[:, _H:2 * _H])
    gg = jnp.tanh(g[:, 2 * _H:3 * _H])
    o = jax.nn.sigmoid(g[:, 3 * _H:4 * _H])
    cn = f * c + i * gg
    hn = o * jnp.tanh(cn)
    em = jnp.dot(hn.astype(jnp.bfloat16), wo_ref[d],
                 preferred_element_type=jnp.float32)
    return hn, cn, em


def _lstm_body(xf_ref, xb_ref, wih_ref, whh_ref, b_ref, wo_ref, h0_ref,
               c0_ref, emf_ref, emb_ref, h_scr, c_scr, gxf_scr, gxb_scr):
    ci = pl.program_id(0)

    @pl.when(ci == 0)
    def _():
        h_scr[...] = h0_ref[...]
        c_scr[...] = c0_ref[...]

    gxf_scr[...] = jnp.dot(xf_ref[0].reshape(_C * _B, _E), wih_ref[0],
                           preferred_element_type=jnp.float32)
    gxb_scr[...] = jnp.dot(xb_ref[0].reshape(_C * _B, _E), wih_ref[1],
                           preferred_element_type=jnp.float32)

    hf, cf = h_scr[0], c_scr[0]
    hb, cb = h_scr[1], c_scr[1]
    for k in range(_C):
        hf, cf, emf = _cell(gxf_scr, k * _B, hf, cf, whh_ref, b_ref, wo_ref, 0)
        emf_ref[k] = emf
        kb = _C - 1 - k
        hb, cb, emb = _cell(gxb_scr, kb * _B, hb, cb, whh_ref, b_ref, wo_ref, 1)
        emb_ref[kb] = emb
    h_scr[0] = hf
    c_scr[0] = cf
    h_scr[1] = hb
    c_scr[1] = cb


def _crf_body(emf_ref, emb_ref, tags_ref, lens_ref, bout_ref, start_ref,
              end_ref, transn_ref, transz_ref, out_ref,
              q_scr, sp_scr, l_scr, acc_scr, poh_scr, expt_scr):
    ci = pl.program_id(0)

    @pl.when(ci == 0)
    def _():
        expt_scr[...] = jnp.exp(transn_ref[...]).astype(jnp.bfloat16)

    lens = lens_ref[...]
    lanes = jax.lax.broadcasted_iota(jnp.int32, (_B, _KP), 1)
    endv = end_ref[...]
    startv = start_ref[...]
    expt = expt_scr[...]
    transz = transz_ref[...]

    q = q_scr[...]
    sp = sp_scr[...]
    ll = l_scr[...]
    acc = acc_scr[...]
    poh = poh_scr[...]

    for k in range(_C):
        t = ci * _C + k
        em_t = emf_ref[k] + emb_ref[k] + bout_ref[...]
        e_t = jnp.exp(em_t)
        oh = (tags_ref[k] == lanes).astype(jnp.float32)
        m = t < lens

        cand = jnp.dot(q.astype(jnp.bfloat16), expt,
                       preferred_element_type=jnp.float32) * e_t
        qn = jnp.where(m, cand, q) * (1.0 / sp)
        lln = ll + jnp.log(sp)
        trow = jnp.dot(poh, transz, preferred_element_type=jnp.float32)
        accn = acc + m.astype(jnp.float32) * oh * (em_t + trow)

        if k == 0:
            first = ci == 0
            q = jnp.where(first, e_t * jnp.exp(startv), qn)
            ll = jnp.where(first, 0.0, lln)
            acc = jnp.where(first, oh * (em_t + startv), accn)
        else:
            q, ll, acc = qn, lln, accn
        poh = oh.astype(jnp.bfloat16)

        acc = acc + jnp.where(lens == t + 1, oh * endv, 0.0)
        sp = jnp.sum(q, axis=-1, keepdims=True)

    q_scr[...] = q
    sp_scr[...] = sp
    l_scr[...] = ll
    acc_scr[...] = acc
    poh_scr[...] = poh

    @pl.when(ci == _TC - 1)
    def _():
        z = jnp.sum(q * jnp.exp(endv), axis=-1, keepdims=True)
        logz = jnp.log(z) + ll
        numer = jnp.sum(acc, axis=-1, keepdims=True)
        total = jnp.sum(numer - logz)
        out_ref[...] = jnp.broadcast_to(total, (1, _KP))


def kernel(embeds, tag_ids, lengths, h0, c0, w_ih_f, w_hh_f, b_ih_f, b_hh_f,
           w_ih_b, w_hh_b, b_ih_b, b_hh_b, w_out, b_out,
           start_trans, end_trans, trans):
    f32 = jnp.float32
    bf16 = jnp.bfloat16

    xT = jnp.transpose(embeds, (1, 0, 2)).astype(bf16)
    xT = xT.reshape(_TC, _C, _B, _E)
    wih = jnp.stack([w_ih_f.T, w_ih_b.T]).astype(bf16)
    whh = jnp.stack([w_hh_f.T, w_hh_b.T]).astype(bf16)
    bias = jnp.stack([b_ih_f + b_hh_f, b_ih_b + b_hh_b])[:, None, :]
    wo = jnp.zeros((2, _H, _KP), f32)
    wo = wo.at[0, :, :_K].set(w_out[:, :_H].T)
    wo = wo.at[1, :, :_K].set(w_out[:, _H:].T)
    wo = wo.astype(bf16)

    emf, emb = pl.pallas_call(
        _lstm_body,
        grid=(_TC,),
        in_specs=[
            pl.BlockSpec((1, _C, _B, _E), lambda c: (c, 0, 0, 0)),
            pl.BlockSpec((1, _C, _B, _E), lambda c: (_TC - 1 - c, 0, 0, 0)),
            pl.BlockSpec((2, _E, _G), lambda c: (0, 0, 0)),
            pl.BlockSpec((2, _H, _G), lambda c: (0, 0, 0)),
            pl.BlockSpec((2, 1, _G), lambda c: (0, 0, 0)),
            pl.BlockSpec((2, _H, _KP), lambda c: (0, 0, 0)),
            pl.BlockSpec((2, _B, _H), lambda c: (0, 0, 0)),
            pl.BlockSpec((2, _B, _H), lambda c: (0, 0, 0)),
        ],
        out_specs=[
            pl.BlockSpec((_C, _B, _KP), lambda c: (c, 0, 0)),
            pl.BlockSpec((_C, _B, _KP), lambda c: (_TC - 1 - c, 0, 0)),
        ],
        out_shape=[
            jax.ShapeDtypeStruct((_T, _B, _KP), f32),
            jax.ShapeDtypeStruct((_T, _B, _KP), f32),
        ],
        scratch_shapes=[pltpu.VMEM((2, _B, _H), f32),
                        pltpu.VMEM((2, _B, _H), f32),
                        pltpu.VMEM((_C * _B, _G), f32),
                        pltpu.VMEM((_C * _B, _G), f32)],
        compiler_params=pltpu.CompilerParams(
            dimension_semantics=("arbitrary",),
            vmem_limit_bytes=56 * 1024 * 1024),
        name="bilstm_em",
    )(xT, xT, wih, whh, bias, wo, h0, c0)

    tags_b = jnp.broadcast_to(tag_ids.T[:, :, None], (_T, _B, _KP))
    lens_b = jnp.broadcast_to(lengths[:, None], (_B, _KP))
    bout_p = jnp.full((1, _KP), _NEG, f32).at[0, :_K].set(b_out)
    start_p = jnp.full((1, _KP), _NEG, f32).at[0, :_K].set(start_trans)
    end_p = jnp.full((1, _KP), _NEG, f32).at[0, :_K].set(end_trans)
    trans_n = jnp.full((_KP, _KP), _NEG, f32).at[:_K, :_K].set(trans)
    trans_z = jnp.zeros((_KP, _KP), bf16).at[:_K, :_K].set(trans.astype(bf16))

    partial = pl.pallas_call(
        _crf_body,
        grid=(_TC,),
        in_specs=[
            pl.BlockSpec((_C, _B, _KP), lambda c: (c, 0, 0)),
            pl.BlockSpec((_C, _B, _KP), lambda c: (c, 0, 0)),
            pl.BlockSpec((_C, _B, _KP), lambda c: (c, 0, 0)),
            pl.BlockSpec((_B, _KP), lambda c: (0, 0)),
            pl.BlockSpec((1, _KP), lambda c: (0, 0)),
            pl.BlockSpec((1, _KP), lambda c: (0, 0)),
            pl.BlockSpec((1, _KP), lambda c: (0, 0)),
            pl.BlockSpec((_KP, _KP), lambda c: (0, 0)),
            pl.BlockSpec((_KP, _KP), lambda c: (0, 0)),
        ],
        out_specs=pl.BlockSpec((1, _KP), lambda c: (0, 0)),
        out_shape=jax.ShapeDtypeStruct((1, _KP), f32),
        scratch_shapes=[
            pltpu.VMEM((_B, _KP), f32),
            pltpu.VMEM((_B, 1), f32),
            pltpu.VMEM((_B, 1), f32),
            pltpu.VMEM((_B, _KP), f32),
            pltpu.VMEM((_B, _KP), jnp.bfloat16),
            pltpu.VMEM((_KP, _KP), jnp.bfloat16),
        ],
        compiler_params=pltpu.CompilerParams(
            dimension_semantics=("arbitrary",)),
        name="crf_nll",
    )(emf, emb, tags_b, lens_b, bout_p, start_p, end_p, trans_n, trans_z)

    return -partial[0, 0] / _B


# speedup vs baseline: 12.1264x; 1.0074x over previous
import jax
import jax.numpy as jnp
from jax.experimental import pallas as pl
from jax.experimental.pallas import tpu as pltpu

_T = 512
_B = 64
_E = 1024
_H = 512
_G = 4 * _H
_K = 74
_KP = 128
_C = 8
_TC = _T // _C
_NEG = -1e30


def _cell(gx_scr, row, h, c, whh_ref, b_ref, wo_ref, d):
    g = jnp.dot(h.astype(jnp.bfloat16), whh_ref[d],
                preferred_element_type=jnp.float32)
    g = g + gx_scr[row:row + _B, :].astype(jnp.float32)
    g = g + b_ref[d]
    i = jax.nn.sigmoid(g[:, 0:_H])
    f = jax.nn.sigmoid(g[:, _H:2 * _H])
    gg = jnp.tanh(g[:, 2 * _H:3 * _H])
    o = jax.nn.sigmoid(g[:, 3 * _H:4 * _H])
    cn = f * c + i * gg
    hn = o * jnp.tanh(cn)
    em = jnp.dot(hn.astype(jnp.bfloat16), wo_ref[d],
                 preferred_element_type=jnp.float32).astype(jnp.bfloat16)
    return hn, cn, em


def _lstm_body(xf_ref, xb_ref, wih_ref, whh_ref, b_ref, wo_ref, h0_ref,
               c0_ref, emf_ref, emb_ref, h_scr, c_scr, gxf_scr, gxb_scr):
    ci = pl.program_id(0)

    @pl.when(ci == 0)
    def _():
        h_scr[...] = h0_ref[...]
        c_scr[...] = c0_ref[...]

    gxf_scr[...] = jnp.dot(xf_ref[0].reshape(_C * _B, _E), wih_ref[0],
                           preferred_element_type=jnp.float32
                           ).astype(jnp.bfloat16)
    gxb_scr[...] = jnp.dot(xb_ref[0].reshape(_C * _B, _E), wih_ref[1],
                           preferred_element_type=jnp.float32
                           ).astype(jnp.bfloat16)

    hf, cf = h_scr[0], c_scr[0]
    hb, cb = h_scr[1], c_scr[1]
    for k in range(_C):
        hf, cf, emf = _cell(gxf_scr, k * _B, hf, cf, whh_ref, b_ref, wo_ref, 0)
        emf_ref[k] = emf
        kb = _C - 1 - k
        hb, cb, emb = _cell(gxb_scr, kb * _B, hb, cb, whh_ref, b_ref, wo_ref, 1)
        emb_ref[kb] = emb
    h_scr[0] = hf
    c_scr[0] = cf
    h_scr[1] = hb
    c_scr[1] = cb


def _crf_body(emf_ref, emb_ref, tags_ref, lens_ref, bout_ref, start_ref,
              end_ref, transn_ref, transz_ref, out_ref,
              q_scr, sp_scr, l_scr, acc_scr, poh_scr, expt_scr):
    ci = pl.program_id(0)

    @pl.when(ci == 0)
    def _():
        expt_scr[...] = jnp.exp(transn_ref[...]).astype(jnp.bfloat16)

    lens = lens_ref[...]
    lanes = jax.lax.broadcasted_iota(jnp.int32, (_B, _KP), 1)
    endv = end_ref[...]
    startv = start_ref[...]
    expt = expt_scr[...]
    transz = transz_ref[...]
    tagsT = jnp.swapaxes(tags_ref[0], 0, 1)

    q = q_scr[...]
    sp = sp_scr[...]
    ll = l_scr[...]
    acc = acc_scr[...]
    poh = poh_scr[...]

    for k in range(_C):
        t = ci * _C + k
        em_t = (emf_ref[k].astype(jnp.float32)
                + emb_ref[k].astype(jnp.float32) + bout_ref[...])
        e_t = jnp.exp(em_t)
        oh = (tagsT[:, k:k + 1] == lanes).astype(jnp.float32)
        m = t < lens

        cand = jnp.dot(q.astype(jnp.bfloat16), expt,
                       preferred_element_type=jnp.float32) * e_t
        qn = jnp.where(m, cand, q) * (1.0 / sp)
        lln = ll + jnp.log(sp)
        trow = jnp.dot(poh, transz, preferred_element_type=jnp.float32)
        accn = acc + m.astype(jnp.float32) * oh * (em_t + trow)

        if k == 0:
            first = ci == 0
            q = jnp.where(first, e_t * jnp.exp(startv), qn)
            ll = jnp.where(first, 0.0, lln)
            acc = jnp.where(first, oh * (em_t + startv), accn)
        else:
            q, ll, acc = qn, lln, accn
        poh = oh.astype(jnp.bfloat16)

        acc = acc + jnp.where(lens == t + 1, oh * endv, 0.0)
        sp = jnp.sum(q, axis=-1, keepdims=True)

    q_scr[...] = q
    sp_scr[...] = sp
    l_scr[...] = ll
    acc_scr[...] = acc
    poh_scr[...] = poh

    @pl.when(ci == _TC - 1)
    def _():
        z = jnp.sum(q * jnp.exp(endv), axis=-1, keepdims=True)
        logz = jnp.log(z) + ll
        numer = jnp.sum(acc, axis=-1, keepdims=True)
        total = jnp.sum(numer - logz)
        out_ref[...] = jnp.broadcast_to(total, (1, _KP))


def kernel(embeds, tag_ids, lengths, h0, c0, w_ih_f, w_hh_f, b_ih_f, b_hh_f,
           w_ih_b, w_hh_b, b_ih_b, b_hh_b, w_out, b_out,
           start_trans, end_trans, trans):
    f32 = jnp.float32
    bf16 = jnp.bfloat16

    xT = jnp.transpose(embeds.astype(bf16), (1, 0, 2))
    xT = xT.reshape(_TC, _C, _B, _E)
    wih = jnp.stack([w_ih_f.T, w_ih_b.T]).astype(bf16)
    whh = jnp.stack([w_hh_f.T, w_hh_b.T]).astype(bf16)
    bias = jnp.stack([b_ih_f + b_hh_f, b_ih_b + b_hh_b])[:, None, :]
    wo = jnp.zeros((2, _H, _KP), f32)
    wo = wo.at[0, :, :_K].set(w_out[:, :_H].T)
    wo = wo.at[1, :, :_K].set(w_out[:, _H:].T)
    wo = wo.astype(bf16)

    emf, emb = pl.pallas_call(
        _lstm_body,
        grid=(_TC,),
        in_specs=[
            pl.BlockSpec((1, _C, _B, _E), lambda c: (c, 0, 0, 0)),
            pl.BlockSpec((1, _C, _B, _E), lambda c: (_TC - 1 - c, 0, 0, 0)),
            pl.BlockSpec((2, _E, _G), lambda c: (0, 0, 0)),
            pl.BlockSpec((2, _H, _G), lambda c: (0, 0, 0)),
            pl.BlockSpec((2, 1, _G), lambda c: (0, 0, 0)),
            pl.BlockSpec((2, _H, _KP), lambda c: (0, 0, 0)),
            pl.BlockSpec((2, _B, _H), lambda c: (0, 0, 0)),
            pl.BlockSpec((2, _B, _H), lambda c: (0, 0, 0)),
        ],
        out_specs=[
            pl.BlockSpec((_C, _B, _KP), lambda c: (c, 0, 0)),
            pl.BlockSpec((_C, _B, _KP), lambda c: (_TC - 1 - c, 0, 0)),
        ],
        out_shape=[
            jax.ShapeDtypeStruct((_T, _B, _KP), bf16),
            jax.ShapeDtypeStruct((_T, _B, _KP), bf16),
        ],
        scratch_shapes=[pltpu.VMEM((2, _B, _H), f32),
                        pltpu.VMEM((2, _B, _H), f32),
                        pltpu.VMEM((_C * _B, _G), bf16),
                        pltpu.VMEM((_C * _B, _G), bf16)],
        compiler_params=pltpu.CompilerParams(
            dimension_semantics=("arbitrary",),
            vmem_limit_bytes=56 * 1024 * 1024),
        name="bilstm_em",
    )(xT, xT, wih, whh, bias, wo, h0, c0)

    tags_c = tag_ids.T.reshape(_TC, _C, _B)
    lens_b = jnp.broadcast_to(lengths[:, None], (_B, _KP))
    bout_p = jnp.full((1, _KP), _NEG, f32).at[0, :_K].set(b_out)
    start_p = jnp.full((1, _KP), _NEG, f32).at[0, :_K].set(start_trans)
    end_p = jnp.full((1, _KP), _NEG, f32).at[0, :_K].set(end_trans)
    trans_n = jnp.full((_KP, _KP), _NEG, f32).at[:_K, :_K].set(trans)
    trans_z = jnp.zeros((_KP, _KP), bf16).at[:_K, :_K].set(trans.astype(bf16))

    partial = pl.pallas_call(
        _crf_body,
        grid=(_TC,),
        in_specs=[
            pl.BlockSpec((_C, _B, _KP), lambda c: (c, 0, 0)),
            pl.BlockSpec((_C, _B, _KP), lambda c: (c, 0, 0)),
            pl.BlockSpec((1, _C, _B), lambda c: (c, 0, 0)),
            pl.BlockSpec((_B, _KP), lambda c: (0, 0)),
            pl.BlockSpec((1, _KP), lambda c: (0, 0)),
            pl.BlockSpec((1, _KP), lambda c: (0, 0)),
            pl.BlockSpec((1, _KP), lambda c: (0, 0)),
            pl.BlockSpec((_KP, _KP), lambda c: (0, 0)),
            pl.BlockSpec((_KP, _KP), lambda c: (0, 0)),
        ],
        out_specs=pl.BlockSpec((1, _KP), lambda c: (0, 0)),
        out_shape=jax.ShapeDtypeStruct((1, _KP), f32),
        scratch_shapes=[
            pltpu.VMEM((_B, _KP), f32),
            pltpu.VMEM((_B, 1), f32),
            pltpu.VMEM((_B, 1), f32),
            pltpu.VMEM((_B, _KP), f32),
            pltpu.VMEM((_B, _KP), jnp.bfloat16),
            pltpu.VMEM((_KP, _KP), jnp.bfloat16),
        ],
        compiler_params=pltpu.CompilerParams(
            dimension_semantics=("arbitrary",)),
        name="crf_nll",
    )(emf, emb, tags_c, lens_b, bout_p, start_p, end_p, trans_n, trans_z)

    return -partial[0, 0] / _B


# speedup vs baseline: 13.4266x; 1.1072x over previous
import jax
import jax.numpy as jnp
from jax.experimental import pallas as pl
from jax.experimental.pallas import tpu as pltpu

_T = 512
_B = 64
_E = 1024
_H = 512
_G = 4 * _H
_K = 74
_KP = 128
_C = 8
_TC = _T // _C
_NEG = -1e30


def _cell(gx_scr, row, h, c, whh_ref, b_ref, wo_ref, d):
    g = jnp.dot(h.astype(jnp.bfloat16), whh_ref[d],
                preferred_element_type=jnp.float32)
    g = g + gx_scr[row:row + _B, :].astype(jnp.float32)
    g = g + b_ref[d]
    i = jax.nn.sigmoid(g[:, 0:_H])
    f = jax.nn.sigmoid(g[:, _H:2 * _H])
    gg = jnp.tanh(g[:, 2 * _H:3 * _H])
    o = jax.nn.sigmoid(g[:, 3 * _H:4 * _H])
    cn = f * c + i * gg
    hn = o * jnp.tanh(cn)
    em = jnp.dot(hn.astype(jnp.bfloat16), wo_ref[d],
                 preferred_element_type=jnp.float32).astype(jnp.bfloat16)
    return hn, cn, em


def _lstm_body(xf_ref, xb_ref, wih_ref, whh_ref, b_ref, wo_ref, h0_ref,
               c0_ref, emf_ref, emb_ref, h_scr, c_scr, gxf_scr, gxb_scr):
    ci = pl.program_id(0)

    @pl.when(ci == 0)
    def _():
        h_scr[...] = h0_ref[...]
        c_scr[...] = c0_ref[...]

    xf = jnp.swapaxes(xf_ref[:, 0].astype(jnp.bfloat16), 0, 1)
    xb = jnp.swapaxes(xb_ref[:, 0].astype(jnp.bfloat16), 0, 1)
    gxf_scr[...] = jnp.dot(xf.reshape(_C * _B, _E), wih_ref[0],
                           preferred_element_type=jnp.float32
                           ).astype(jnp.bfloat16)
    gxb_scr[...] = jnp.dot(xb.reshape(_C * _B, _E), wih_ref[1],
                           preferred_element_type=jnp.float32
                           ).astype(jnp.bfloat16)

    hf, cf = h_scr[0], c_scr[0]
    hb, cb = h_scr[1], c_scr[1]
    for k in range(_C):
        hf, cf, emf = _cell(gxf_scr, k * _B, hf, cf, whh_ref, b_ref, wo_ref, 0)
        emf_ref[k] = emf
        kb = _C - 1 - k
        hb, cb, emb = _cell(gxb_scr, kb * _B, hb, cb, whh_ref, b_ref, wo_ref, 1)
        emb_ref[kb] = emb
    h_scr[0] = hf
    c_scr[0] = cf
    h_scr[1] = hb
    c_scr[1] = cb


def _crf_body(emf_ref, emb_ref, tags_ref, lens_ref, bout_ref, start_ref,
              end_ref, transn_ref, transz_ref, out_ref,
              q_scr, sp_scr, l_scr, acc_scr, poh_scr, expt_scr):
    ci = pl.program_id(0)

    @pl.when(ci == 0)
    def _():
        expt_scr[...] = jnp.exp(transn_ref[...]).astype(jnp.bfloat16)

    lens = lens_ref[...]
    lanes = jax.lax.broadcasted_iota(jnp.int32, (_B, _KP), 1)
    endv = end_ref[...]
    startv = start_ref[...]
    expt = expt_scr[...]
    transz = transz_ref[...]
    tagsT = jnp.swapaxes(tags_ref[0], 0, 1)

    q = q_scr[...]
    sp = sp_scr[...]
    ll = l_scr[...]
    acc = acc_scr[...]
    poh = poh_scr[...]

    for k in range(_C):
        t = ci * _C + k
        em_t = (emf_ref[k].astype(jnp.float32)
                + emb_ref[k].astype(jnp.float32) + bout_ref[...])
        e_t = jnp.exp(em_t)
        oh = (tagsT[:, k:k + 1] == lanes).astype(jnp.float32)
        m = t < lens

        cand = jnp.dot(q.astype(jnp.bfloat16), expt,
                       preferred_element_type=jnp.float32) * e_t
        qn = jnp.where(m, cand, q) * (1.0 / sp)
        lln = ll + jnp.log(sp)
        trow = jnp.dot(poh, transz, preferred_element_type=jnp.float32)
        accn = acc + m.astype(jnp.float32) * oh * (em_t + trow)

        if k == 0:
            first = ci == 0
            q = jnp.where(first, e_t * jnp.exp(startv), qn)
            ll = jnp.where(first, 0.0, lln)
            acc = jnp.where(first, oh * (em_t + startv), accn)
        else:
            q, ll, acc = qn, lln, accn
        poh = oh.astype(jnp.bfloat16)

        acc = acc + jnp.where(lens == t + 1, oh * endv, 0.0)
        sp = jnp.sum(q, axis=-1, keepdims=True)

    q_scr[...] = q
    sp_scr[...] = sp
    l_scr[...] = ll
    acc_scr[...] = acc
    poh_scr[...] = poh

    @pl.when(ci == _TC - 1)
    def _():
        z = jnp.sum(q * jnp.exp(endv), axis=-1, keepdims=True)
        logz = jnp.log(z) + ll
        numer = jnp.sum(acc, axis=-1, keepdims=True)
        total = jnp.sum(numer - logz)
        out_ref[...] = jnp.broadcast_to(total, (1, _KP))


def kernel(embeds, tag_ids, lengths, h0, c0, w_ih_f, w_hh_f, b_ih_f, b_hh_f,
           w_ih_b, w_hh_b, b_ih_b, b_hh_b, w_out, b_out,
           start_trans, end_trans, trans):
    f32 = jnp.float32
    bf16 = jnp.bfloat16

    xBT = embeds.reshape(_B, _TC, _C, _E)
    wih = jnp.stack([w_ih_f.T, w_ih_b.T]).astype(bf16)
    whh = jnp.stack([w_hh_f.T, w_hh_b.T]).astype(bf16)
    bias = jnp.stack([b_ih_f + b_hh_f, b_ih_b + b_hh_b])[:, None, :]
    wo = jnp.zeros((2, _H, _KP), f32)
    wo = wo.at[0, :, :_K].set(w_out[:, :_H].T)
    wo = wo.at[1, :, :_K].set(w_out[:, _H:].T)
    wo = wo.astype(bf16)

    emf, emb = pl.pallas_call(
        _lstm_body,
        grid=(_TC,),
        in_specs=[
            pl.BlockSpec((_B, 1, _C, _E), lambda c: (0, c, 0, 0)),
            pl.BlockSpec((_B, 1, _C, _E), lambda c: (0, _TC - 1 - c, 0, 0)),
            pl.BlockSpec((2, _E, _G), lambda c: (0, 0, 0)),
            pl.BlockSpec((2, _H, _G), lambda c: (0, 0, 0)),
            pl.BlockSpec((2, 1, _G), lambda c: (0, 0, 0)),
            pl.BlockSpec((2, _H, _KP), lambda c: (0, 0, 0)),
            pl.BlockSpec((2, _B, _H), lambda c: (0, 0, 0)),
            pl.BlockSpec((2, _B, _H), lambda c: (0, 0, 0)),
        ],
        out_specs=[
            pl.BlockSpec((_C, _B, _KP), lambda c: (c, 0, 0)),
            pl.BlockSpec((_C, _B, _KP), lambda c: (_TC - 1 - c, 0, 0)),
        ],
        out_shape=[
            jax.ShapeDtypeStruct((_T, _B, _KP), bf16),
            jax.ShapeDtypeStruct((_T, _B, _KP), bf16),
        ],
        scratch_shapes=[pltpu.VMEM((2, _B, _H), f32),
                        pltpu.VMEM((2, _B, _H), f32),
                        pltpu.VMEM((_C * _B, _G), bf16),
                        pltpu.VMEM((_C * _B, _G), bf16)],
        compiler_params=pltpu.CompilerParams(
            dimension_semantics=("arbitrary",),
            vmem_limit_bytes=56 * 1024 * 1024),
        name="bilstm_em",
    )(xBT, xBT, wih, whh, bias, wo, h0, c0)

    tags_c = tag_ids.T.reshape(_TC, _C, _B)
    lens_b = jnp.broadcast_to(lengths[:, None], (_B, _KP))
    bout_p = jnp.full((1, _KP), _NEG, f32).at[0, :_K].set(b_out)
    start_p = jnp.full((1, _KP), _NEG, f32).at[0, :_K].set(start_trans)
    end_p = jnp.full((1, _KP), _NEG, f32).at[0, :_K].set(end_trans)
    trans_n = jnp.full((_KP, _KP), _NEG, f32).at[:_K, :_K].set(trans)
    trans_z = jnp.zeros((_KP, _KP), bf16).at[:_K, :_K].set(trans.astype(bf16))

    partial = pl.pallas_call(
        _crf_body,
        grid=(_TC,),
        in_specs=[
            pl.BlockSpec((_C, _B, _KP), lambda c: (c, 0, 0)),
            pl.BlockSpec((_C, _B, _KP), lambda c: (c, 0, 0)),
            pl.BlockSpec((1, _C, _B), lambda c: (c, 0, 0)),
            pl.BlockSpec((_B, _KP), lambda c: (0, 0)),
            pl.BlockSpec((1, _KP), lambda c: (0, 0)),
            pl.BlockSpec((1, _KP), lambda c: (0, 0)),
            pl.BlockSpec((1, _KP), lambda c: (0, 0)),
            pl.BlockSpec((_KP, _KP), lambda c: (0, 0)),
            pl.BlockSpec((_KP, _KP), lambda c: (0, 0)),
        ],
        out_specs=pl.BlockSpec((1, _KP), lambda c: (0, 0)),
        out_shape=jax.ShapeDtypeStruct((1, _KP), f32),
        scratch_shapes=[
            pltpu.VMEM((_B, _KP), f32),
            pltpu.VMEM((_B, 1), f32),
            pltpu.VMEM((_B, 1), f32),
            pltpu.VMEM((_B, _KP), f32),
            pltpu.VMEM((_B, _KP), jnp.bfloat16),
            pltpu.VMEM((_KP, _KP), jnp.bfloat16),
        ],
        compiler_params=pltpu.CompilerParams(
            dimension_semantics=("arbitrary",)),
        name="crf_nll",
    )(emf, emb, tags_c, lens_b, bout_p, start_p, end_p, trans_n, trans_z)

    return -partial[0, 0] / _B
